# baseline (speedup 1.0000x reference)
import functools

import jax
import jax.numpy as jnp
from jax import lax
from jax.experimental import pallas as pl
from jax.experimental.pallas import tpu as pltpu
from jax.experimental.pallas import tpu_sc as plsc

VOCAB = 100000
EMBED = 32
BATCH = 1024


_info = plsc.get_sparse_core_info()
_NC, _NS = _info.num_cores, _info.num_subcores
_NW = _NC * _NS
_L = _info.num_lanes
_NGRP = BATCH // _L


def _sc_gather_t(table_t, ids):
  mesh = plsc.VectorSubcoreMesh(core_axis_name="c", subcore_axis_name="s")

  @functools.partial(
      pl.kernel,
      mesh=mesh,
      out_type=jax.ShapeDtypeStruct((EMBED, BATCH), jnp.float32),
      scratch_types=[
          pltpu.VMEM((BATCH,), jnp.int32),
          pltpu.VMEM((VOCAB,), jnp.float32),
          pltpu.VMEM((BATCH,), jnp.float32),
          pltpu.SemaphoreType.DMA,
      ],
      compiler_params=pltpu.CompilerParams(needs_layout_passes=False),
  )
  def gather_kernel(table_hbm, idx_hbm, out_hbm, idx_v, row_v, xt_v, sem):
    wid = lax.axis_index("s") * _NC + lax.axis_index("c")
    cp_row = pltpu.async_copy(table_hbm.at[wid], row_v, sem)
    pltpu.sync_copy(idx_hbm, idx_v)
    cp_row.wait()
    for g in range(_NGRP):
      idx16 = idx_v[pl.ds(g * _L, _L)]
      xt_v[pl.ds(g * _L, _L)] = plsc.load_gather(row_v, [idx16])
    pltpu.sync_copy(xt_v, out_hbm.at[wid])

  return gather_kernel(table_t, ids)


_VBLK = 5120
_NBLK = -(-VOCAB // _VBLK)


def _proj_body(xt_ref, wt_ref, b_ref, out_ref):
  acc = lax.dot_general(
      wt_ref[...], xt_ref[...],
      dimension_numbers=(((0,), (0,)), ((), ())),
      preferred_element_type=jnp.float32,
  )
  bias = lax.broadcast_in_dim(b_ref[0, :], (_VBLK, BATCH), (0,))
  out_ref[...] = acc + bias


def _tc_project_t(xt, w_t, bias_2d):
  return pl.pallas_call(
      _proj_body,
      grid=(_NBLK,),
      in_specs=[
          pl.BlockSpec((EMBED, BATCH), lambda j: (0, 0)),
          pl.BlockSpec((EMBED, _VBLK), lambda j: (0, j)),
          pl.BlockSpec((1, _VBLK), lambda j: (0, j)),
      ],
      out_specs=pl.BlockSpec((_VBLK, BATCH), lambda j: (j, 0)),
      out_shape=jax.ShapeDtypeStruct((VOCAB, BATCH), jnp.float32),
  )(xt, w_t, bias_2d)


def kernel(token_id, embedding_weight, linear_weight, linear_bias):
  ids = token_id.astype(jnp.int32)
  xt = _sc_gather_t(embedding_weight.T, ids)
  logits_t = _tc_project_t(xt, linear_weight.T, linear_bias.reshape(1, VOCAB))
  loss = jnp.array(0.0, dtype=jnp.float32)
  return (logits_t.T, loss)


# speedup vs baseline: 1.0141x; 1.0141x over previous
import functools

import jax
import jax.numpy as jnp
from jax import lax
from jax.experimental import pallas as pl
from jax.experimental.pallas import tpu as pltpu
from jax.experimental.pallas import tpu_sc as plsc

VOCAB = 100000
EMBED = 32
BATCH = 1024


_info = plsc.get_sparse_core_info()
_NC, _NS = _info.num_cores, _info.num_subcores
_NW = _NC * _NS
_L = _info.num_lanes
_NGRP = BATCH // _L


def _sc_gather_t(table_t, ids):
  mesh = plsc.VectorSubcoreMesh(core_axis_name="c", subcore_axis_name="s")

  @functools.partial(
      pl.kernel,
      mesh=mesh,
      out_type=jax.ShapeDtypeStruct((EMBED, BATCH), jnp.float32),
      scratch_types=[
          pltpu.VMEM((BATCH,), jnp.int32),
          pltpu.VMEM((VOCAB,), jnp.float32),
          pltpu.VMEM((BATCH,), jnp.float32),
          pltpu.SemaphoreType.DMA,
      ],
      compiler_params=pltpu.CompilerParams(needs_layout_passes=False),
  )
  def gather_kernel(table_hbm, idx_hbm, out_hbm, idx_v, row_v, xt_v, sem):
    wid = lax.axis_index("s") * _NC + lax.axis_index("c")
    cp_row = pltpu.async_copy(table_hbm.at[wid], row_v, sem)
    pltpu.sync_copy(idx_hbm, idx_v)
    cp_row.wait()
    for g in range(_NGRP):
      idx16 = idx_v[pl.ds(g * _L, _L)]
      xt_v[pl.ds(g * _L, _L)] = plsc.load_gather(row_v, [idx16])
    pltpu.sync_copy(xt_v, out_hbm.at[wid])

  return gather_kernel(table_t, ids)


_VBLK = 3200
_NBLK = -(-VOCAB // _VBLK)


def _proj_body(xt_ref, wt_ref, b_ref, out_ref):
  acc = lax.dot_general(
      wt_ref[...], xt_ref[...],
      dimension_numbers=(((0,), (0,)), ((), ())),
      preferred_element_type=jnp.float32,
  )
  bias = lax.broadcast_in_dim(b_ref[0, :], (_VBLK, BATCH), (0,))
  out_ref[...] = acc + bias


def _tc_project_t(xt, w_t, bias_2d):
  return pl.pallas_call(
      _proj_body,
      grid=(_NBLK,),
      in_specs=[
          pl.BlockSpec((EMBED, BATCH), lambda j: (0, 0)),
          pl.BlockSpec((EMBED, _VBLK), lambda j: (0, j)),
          pl.BlockSpec((1, _VBLK), lambda j: (0, j)),
      ],
      out_specs=pl.BlockSpec((_VBLK, BATCH), lambda j: (j, 0)),
      out_shape=jax.ShapeDtypeStruct((VOCAB, BATCH), jnp.float32),
  )(xt, w_t, bias_2d)


def kernel(token_id, embedding_weight, linear_weight, linear_bias):
  ids = token_id.astype(jnp.int32)
  xt = _sc_gather_t(embedding_weight.T, ids)
  logits_t = _tc_project_t(xt, linear_weight.T, linear_bias.reshape(1, VOCAB))
  loss = jnp.array(0.0, dtype=jnp.float32)
  return (logits_t.T, loss)
